# baseline (speedup 1.0000x reference)
import functools

import jax
import jax.numpy as jnp
from jax.experimental import pallas as pl
from jax.experimental.pallas import tpu as pltpu


def _mega_kernel(ns_ref, labels_ref, emb_ref, wef_ref, bef_ref,
                 Wz_ref, Wr_ref, Wh_ref, bz_ref, br_ref, bh_ref,
                 adj_ref, h_ref, adj16_ref,
                 inb, obb, bfb, msgs_ref, in_sems, out_sems,
                 *, bm1, bm, nt, d, t_fwd):
    n = h_ref.shape[0]
    nb1 = n // bm1
    nb = n // bm

    def in_copy_f32(i, slot, t):
        return pltpu.make_async_copy(
            adj_ref.at[t, pl.ds(i * bm1, bm1), :], inb.at[slot, t],
            in_sems.at[slot, t])

    def out_copy(i, slot, t):
        return pltpu.make_async_copy(
            obb.at[slot, t], adj16_ref.at[t, pl.ds(i * bm1, bm1), :],
            out_sems.at[slot, t])

    def in_copy_bf16(i, slot, t):
        return pltpu.make_async_copy(
            adj16_ref.at[t, pl.ds(i * bm, bm), :], bfb.at[slot, t],
            in_sems.at[slot, t])

    for t in range(nt):
        in_copy_f32(0, 0, t).start()

    lab = labels_ref[:]
    iota = jax.lax.broadcasted_iota(jnp.int32, (n, emb_ref.shape[0]), 1)
    onehot = (lab == iota).astype(jnp.float32)
    h_ref[:] = jnp.dot(onehot, emb_ref[:], preferred_element_type=jnp.float32)

    def refresh_msgs():
        m = (jnp.dot(h_ref[:], wef_ref[:], preferred_element_type=jnp.float32)
             + bef_ref[:])
        for t in range(nt):
            msgs_ref[t] = m[:, t * d:(t + 1) * d].astype(jnp.bfloat16)

    refresh_msgs()

    def lin3(a, b, c, w_ref, bias_ref):
        return (
            jnp.dot(a, w_ref[0:d, :], preferred_element_type=jnp.float32)
            + jnp.dot(b, w_ref[d:2 * d, :], preferred_element_type=jnp.float32)
            + jnp.dot(c, w_ref[2 * d:3 * d, :], preferred_element_type=jnp.float32)
            + bias_ref[:]
        )

    def accum(acc_in, acc_out, t, slab):
        p = jnp.dot(slab, msgs_ref[t], preferred_element_type=jnp.float32)
        if t == 0:
            return p, acc_out
        if t < t_fwd:
            return acc_in + p, acc_out
        if t == t_fwd:
            return acc_in, p
        return acc_in, acc_out + p

    def gru_update(i, blk, a_in, a_out):
        h_blk = h_ref[i * blk:(i + 1) * blk, :]
        z = jax.nn.sigmoid(lin3(a_in, a_out, h_blk, Wz_ref, bz_ref))
        r = jax.nn.sigmoid(lin3(a_in, a_out, h_blk, Wr_ref, br_ref))
        h_hat = jnp.tanh(lin3(a_in, a_out, r * h_blk, Wh_ref, bh_ref))
        h_ref[i * blk:(i + 1) * blk, :] = (1.0 - z) * h_blk + z * h_hat

    for i in range(nb1):
        b = i % 2
        if i + 1 < nb1:
            for t in range(nt):
                in_copy_f32(i + 1, 1 - b, t).start()
        a_in = a_out = None
        for t in range(nt):
            in_copy_f32(i, b, t).wait()
            if i >= 2:
                out_copy(i - 2, b, t).wait()
            obb[b, t] = inb[b, t].astype(jnp.bfloat16)
            out_copy(i, b, t).start()
            a_in, a_out = accum(a_in, a_out, t, obb[b, t])
        gru_update(i, bm1, a_in, a_out)
    for t in range(nt):
        out_copy(nb1 - 2, nb1 % 2, t).wait()
        out_copy(nb1 - 1, 1 - nb1 % 2, t).wait()

    def step_body(_, carry):
        for t in range(nt):
            in_copy_bf16(0, 0, t).start()
        refresh_msgs()
        for i in range(nb):
            b = i % 2
            if i + 1 < nb:
                for t in range(nt):
                    in_copy_bf16(i + 1, 1 - b, t).start()
            a_in = a_out = None
            for t in range(nt):
                in_copy_bf16(i, b, t).wait()
                a_in, a_out = accum(a_in, a_out, t, bfb[b, t])
            gru_update(i, bm, a_in, a_out)
        return carry

    jax.lax.fori_loop(0, ns_ref[0] - 1, step_body, 0)


def kernel(adj_tensor, node_labels, n_prop_steps, emb, W_edge, b_edge,
           Wz, bz, Wr, br, Wh, bh):
    nt, n, _ = adj_tensor.shape
    d = emb.shape[1]
    bm1 = 128
    bm = 256
    t_fwd = nt // 2

    We_flat = W_edge.transpose(1, 0, 2).reshape(d, nt * d)
    be_flat = b_edge.reshape(1, nt * d)
    labels2d = node_labels.astype(jnp.int32).reshape(n, 1)
    ns = jnp.asarray(n_prop_steps, jnp.int32).reshape(1)

    vmem = lambda: pl.BlockSpec(memory_space=pltpu.VMEM)
    h, _ = pl.pallas_call(
        functools.partial(_mega_kernel, bm1=bm1, bm=bm, nt=nt, d=d,
                          t_fwd=t_fwd),
        in_specs=[
            pl.BlockSpec(memory_space=pltpu.SMEM),
            vmem(),
            vmem(),
            vmem(),
            vmem(),
            vmem(), vmem(), vmem(),
            vmem(), vmem(), vmem(),
            pl.BlockSpec(memory_space=pl.ANY),
        ],
        out_specs=[
            vmem(),
            pl.BlockSpec(memory_space=pl.ANY),
        ],
        out_shape=[
            jax.ShapeDtypeStruct((n, d), jnp.float32),
            jax.ShapeDtypeStruct((nt, n, n), jnp.bfloat16),
        ],
        scratch_shapes=[
            pltpu.VMEM((2, nt, bm1, n), jnp.float32),
            pltpu.VMEM((2, nt, bm1, n), jnp.bfloat16),
            pltpu.VMEM((2, nt, bm, n), jnp.bfloat16),
            pltpu.VMEM((nt, n, d), jnp.bfloat16),
            pltpu.SemaphoreType.DMA((2, nt)),
            pltpu.SemaphoreType.DMA((2, nt)),
        ],
    )(ns, labels2d, emb, We_flat, be_flat, Wz, Wr, Wh,
      bz.reshape(1, d), br.reshape(1, d), bh.reshape(1, d), adj_tensor)
    return h


# speedup vs baseline: 1.1370x; 1.1370x over previous
import functools

import jax
import jax.numpy as jnp
from jax.experimental import pallas as pl
from jax.experimental.pallas import tpu as pltpu


def _mega_kernel(ns_ref, labels_ref, emb_ref, wef_ref, bef_ref,
                 Wz_ref, Wr_ref, Wh_ref, bz_ref, br_ref, bh_ref,
                 adj_ref, h_ref, adj16_ref,
                 inb, bfb, msgs_ref, in_sems, out_sems,
                 *, bm, nt, d, t_fwd):
    n = h_ref.shape[0]
    nb = n // bm

    def in_copy_f32(i, slot):
        return pltpu.make_async_copy(
            adj_ref.at[:, pl.ds(i * bm, bm), :], inb.at[slot],
            in_sems.at[slot])

    def out_copy(i, slot):
        return pltpu.make_async_copy(
            bfb.at[slot], adj16_ref.at[:, pl.ds(i * bm, bm), :],
            out_sems.at[slot])

    def in_copy_bf16(i, slot):
        return pltpu.make_async_copy(
            adj16_ref.at[:, pl.ds(i * bm, bm), :], bfb.at[slot],
            in_sems.at[slot])

    in_copy_f32(0, 0).start()

    lab = labels_ref[:]
    iota = jax.lax.broadcasted_iota(jnp.int32, (n, emb_ref.shape[0]), 1)
    onehot = (lab == iota).astype(jnp.float32)
    h_ref[:] = jnp.dot(onehot, emb_ref[:], preferred_element_type=jnp.float32)

    def refresh_msgs():
        m = (jnp.dot(h_ref[:], wef_ref[:], preferred_element_type=jnp.float32)
             + bef_ref[:])
        for t in range(nt):
            msgs_ref[t] = m[:, t * d:(t + 1) * d].astype(jnp.bfloat16)

    refresh_msgs()

    def lin3(a, b, c, w_ref, bias_ref):
        return (
            jnp.dot(a, w_ref[0:d, :], preferred_element_type=jnp.float32)
            + jnp.dot(b, w_ref[d:2 * d, :], preferred_element_type=jnp.float32)
            + jnp.dot(c, w_ref[2 * d:3 * d, :], preferred_element_type=jnp.float32)
            + bias_ref[:]
        )

    def aggregate(slab):
        a_in = jnp.dot(slab(0), msgs_ref[0], preferred_element_type=jnp.float32)
        for t in range(1, t_fwd):
            a_in += jnp.dot(slab(t), msgs_ref[t],
                            preferred_element_type=jnp.float32)
        a_out = jnp.dot(slab(t_fwd), msgs_ref[t_fwd],
                        preferred_element_type=jnp.float32)
        for t in range(t_fwd + 1, nt):
            a_out += jnp.dot(slab(t), msgs_ref[t],
                             preferred_element_type=jnp.float32)
        return a_in, a_out

    def gru_update(i, a_in, a_out):
        h_blk = h_ref[i * bm:(i + 1) * bm, :]
        z = jax.nn.sigmoid(lin3(a_in, a_out, h_blk, Wz_ref, bz_ref))
        r = jax.nn.sigmoid(lin3(a_in, a_out, h_blk, Wr_ref, br_ref))
        h_hat = jnp.tanh(lin3(a_in, a_out, r * h_blk, Wh_ref, bh_ref))
        h_ref[i * bm:(i + 1) * bm, :] = (1.0 - z) * h_blk + z * h_hat

    for i in range(nb):
        b = i % 2
        if i + 1 < nb:
            in_copy_f32(i + 1, 1 - b).start()
        in_copy_f32(i, b).wait()
        if i >= 2:
            out_copy(i - 2, b).wait()
        for t in range(nt):
            bfb[b, t] = inb[b, t].astype(jnp.bfloat16)
        out_copy(i, b).start()
        a_in, a_out = aggregate(lambda t: bfb[b, t])
        gru_update(i, a_in, a_out)
    out_copy(nb - 2, nb % 2).wait()
    out_copy(nb - 1, 1 - nb % 2).wait()

    def step_body(_, carry):
        in_copy_bf16(0, 0).start()
        refresh_msgs()
        for i in range(nb):
            b = i % 2
            if i + 1 < nb:
                in_copy_bf16(i + 1, 1 - b).start()
            in_copy_bf16(i, b).wait()
            a_in, a_out = aggregate(lambda t: bfb[b, t])
            gru_update(i, a_in, a_out)
        return carry

    jax.lax.fori_loop(0, ns_ref[0] - 1, step_body, 0)


def kernel(adj_tensor, node_labels, n_prop_steps, emb, W_edge, b_edge,
           Wz, bz, Wr, br, Wh, bh):
    nt, n, _ = adj_tensor.shape
    d = emb.shape[1]
    bm = 256
    t_fwd = nt // 2

    We_flat = W_edge.transpose(1, 0, 2).reshape(d, nt * d)
    be_flat = b_edge.reshape(1, nt * d)
    labels2d = node_labels.astype(jnp.int32).reshape(n, 1)
    ns = jnp.asarray(n_prop_steps, jnp.int32).reshape(1)

    vmem = lambda: pl.BlockSpec(memory_space=pltpu.VMEM)
    h, _ = pl.pallas_call(
        functools.partial(_mega_kernel, bm=bm, nt=nt, d=d, t_fwd=t_fwd),
        in_specs=[
            pl.BlockSpec(memory_space=pltpu.SMEM),
            vmem(),
            vmem(),
            vmem(),
            vmem(),
            vmem(), vmem(), vmem(),
            vmem(), vmem(), vmem(),
            pl.BlockSpec(memory_space=pl.ANY),
        ],
        out_specs=[
            vmem(),
            pl.BlockSpec(memory_space=pl.ANY),
        ],
        out_shape=[
            jax.ShapeDtypeStruct((n, d), jnp.float32),
            jax.ShapeDtypeStruct((nt, n, n), jnp.bfloat16),
        ],
        scratch_shapes=[
            pltpu.VMEM((2, nt, bm, n), jnp.float32),
            pltpu.VMEM((2, nt, bm, n), jnp.bfloat16),
            pltpu.VMEM((nt, n, d), jnp.bfloat16),
            pltpu.SemaphoreType.DMA((2,)),
            pltpu.SemaphoreType.DMA((2,)),
        ],
    )(ns, labels2d, emb, We_flat, be_flat, Wz, Wr, Wh,
      bz.reshape(1, d), br.reshape(1, d), bh.reshape(1, d), adj_tensor)
    return h
